# Initial kernel scaffold; baseline (speedup 1.0000x reference)
import jax
import jax.numpy as jnp
from jax.experimental import pallas as pl


def kernel(coords, x_table, y_table):
    raise NotImplementedError("write your pallas kernel here")


# speedup vs baseline: 3.2642x; 3.2642x over previous
import functools

import jax
import jax.numpy as jnp
from jax import lax
from jax.experimental import pallas as pl
from jax.experimental.pallas import tpu as pltpu
from jax.experimental.pallas import tpu_sc as plsc

HALF = 128
B, S = 4096, 50
N = B * S
NC, NS = 2, 16
NW = NC * NS
ROWS_PER_W = N // NW
CHUNK = 128
K = ROWS_PER_W // CHUNK


def _body(idx_hbm, x_hbm, y_hbm, out_hbm, idx_v, obuf, gsem):
  wid = lax.axis_index("s") * NC + lax.axis_index("c")
  pltpu.sync_copy(idx_hbm.at[wid], idx_v)

  @pl.loop(0, K)
  def _(j):
    base = wid * ROWS_PER_W + j * CHUNK
    cx = pltpu.async_copy(x_hbm.at[idx_v.at[0, j]], obuf.at[:, 0], gsem)
    cy = pltpu.async_copy(y_hbm.at[idx_v.at[1, j]], obuf.at[:, 1], gsem)
    cx.wait()
    cy.wait()
    pltpu.sync_copy(obuf, out_hbm.at[pl.ds(base, CHUNK)])


@functools.partial(jax.jit, donate_argnums=())
def _run(idx, x_table, y_table):
  mesh = plsc.VectorSubcoreMesh(core_axis_name="c", subcore_axis_name="s")
  kfn = pl.kernel(
      _body,
      out_type=jax.ShapeDtypeStruct((N, 2, HALF), jnp.float32),
      mesh=mesh,
      scratch_types=[
          pltpu.VMEM((2, K, CHUNK), jnp.int32),
          pltpu.VMEM((CHUNK, 2, HALF), jnp.float32),
          pltpu.SemaphoreType.DMA,
      ],
  )
  return kfn(idx, x_table, y_table)


def kernel(coords, x_table, y_table):
  idx = jnp.asarray(coords, jnp.int32).reshape(N, 2)
  idx = idx.T.reshape(2, NW, K, CHUNK).transpose(1, 0, 2, 3)
  out = _run(idx, x_table, y_table)
  return out.reshape(B, S, 2 * HALF)


# speedup vs baseline: 3.4451x; 1.0554x over previous
import functools

import jax
import jax.numpy as jnp
from jax import lax
from jax.experimental import pallas as pl
from jax.experimental.pallas import tpu as pltpu
from jax.experimental.pallas import tpu_sc as plsc

HALF = 128
B, S = 4096, 50
N = B * S
NC, NS = 2, 16
NW = NC * NS
ROWS_PER_W = N // NW
CHUNK = 128
K = ROWS_PER_W // CHUNK


def _body(idx_hbm, x_hbm, y_hbm, out_hbm, idx_v, obuf, gsem, wsem):
  wid = lax.axis_index("s") * NC + lax.axis_index("c")
  pltpu.sync_copy(idx_hbm.at[wid], idx_v)

  def start_g(j, b):
    pltpu.async_copy(x_hbm.at[idx_v.at[0, j]], obuf.at[b, :, 0], gsem.at[b])
    pltpu.async_copy(y_hbm.at[idx_v.at[1, j]], obuf.at[b, :, 1], gsem.at[b])

  def wait_g(b):
    pltpu.make_async_copy(x_hbm.at[idx_v.at[0, 0]], obuf.at[b, :, 0],
                          gsem.at[b]).wait()
    pltpu.make_async_copy(y_hbm.at[idx_v.at[1, 0]], obuf.at[b, :, 1],
                          gsem.at[b]).wait()

  def start_w(j, b):
    base = wid * ROWS_PER_W + j * CHUNK
    pltpu.async_copy(obuf.at[b], out_hbm.at[pl.ds(base, CHUNK)], wsem.at[b])

  def wait_w(b):
    pltpu.make_async_copy(obuf.at[b], out_hbm.at[pl.ds(0, CHUNK)],
                          wsem.at[b]).wait()

  start_g(0, 0)

  @pl.loop(0, K, step=2)
  def _(j0):
    for b in range(2):
      j = j0 + b

      @pl.when(j >= 1)
      def _():
        wait_w(1 - b)

      @pl.when(j + 1 < K)
      def _():
        start_g(j + 1, 1 - b)

      wait_g(b)
      start_w(j, b)

  wait_w((K - 1) % 2)


@functools.partial(jax.jit, donate_argnums=())
def _run(idx, x_table, y_table):
  mesh = plsc.VectorSubcoreMesh(core_axis_name="c", subcore_axis_name="s")
  kfn = pl.kernel(
      _body,
      out_type=jax.ShapeDtypeStruct((N, 2, HALF), jnp.float32),
      mesh=mesh,
      scratch_types=[
          pltpu.VMEM((2, K, CHUNK), jnp.int32),
          pltpu.VMEM((2, CHUNK, 2, HALF), jnp.float32),
          pltpu.SemaphoreType.DMA((2,)),
          pltpu.SemaphoreType.DMA((2,)),
      ],
  )
  return kfn(idx, x_table, y_table)


def kernel(coords, x_table, y_table):
  idx = jnp.asarray(coords, jnp.int32).reshape(N, 2)
  idx = idx.T.reshape(2, NW, K, CHUNK).transpose(1, 0, 2, 3)
  out = _run(idx, x_table, y_table)
  return out.reshape(B, S, 2 * HALF)


# speedup vs baseline: 4.4238x; 1.2841x over previous
import functools

import jax
import jax.numpy as jnp
from jax import lax
from jax.experimental import pallas as pl
from jax.experimental.pallas import tpu as pltpu
from jax.experimental.pallas import tpu_sc as plsc

HALF = 128
B, S = 4096, 50
N = B * S
NC, NS = 2, 16
NW = NC * NS
BPW = B // NW
ROWS_PER_W = BPW * S
STRIDE = 64
XOFF = BPW * STRIDE
NBUF = 4
PAIRS = 2 * ROWS_PER_W


def _body(idx_hbm, x_hbm, y_hbm, out_hbm, idx_all, idx_u, obuf, gsem, wsem):
  wid = lax.axis_index("s") * NC + lax.axis_index("c")
  pltpu.sync_copy(idx_hbm.at[wid], idx_all)
  lanes = lax.iota(jnp.int32, 16)

  @pl.loop(0, BPW)
  def _(c):
    for p in range(4):
      g = c * S + p * 16 + lanes
      g = jnp.minimum(g, ROWS_PER_W - 1)
      idx_u[pl.ds(c * STRIDE + p * 16, 16)] = plsc.load_gather(
          idx_all, [2 * g])
      idx_u[pl.ds(XOFF + c * STRIDE + p * 16, 16)] = plsc.load_gather(
          idx_all, [2 * g + 1])

  def start_g(c, b):
    pltpu.async_copy(x_hbm.at[idx_u.at[pl.ds(c * STRIDE, S)]],
                     obuf.at[b, :, pl.ds(0, HALF)], gsem.at[b])
    pltpu.async_copy(y_hbm.at[idx_u.at[pl.ds(XOFF + c * STRIDE, S)]],
                     obuf.at[b, :, pl.ds(HALF, HALF)], gsem.at[b])

  def wait_g(b):
    pltpu.make_async_copy(x_hbm.at[idx_u.at[pl.ds(0, S)]],
                          obuf.at[b, :, pl.ds(0, HALF)], gsem.at[b]).wait()
    pltpu.make_async_copy(y_hbm.at[idx_u.at[pl.ds(0, S)]],
                          obuf.at[b, :, pl.ds(HALF, HALF)], gsem.at[b]).wait()

  def start_w(c, b):
    pltpu.async_copy(obuf.at[b], out_hbm.at[wid * BPW + c], wsem.at[b])

  def wait_w(b):
    pltpu.make_async_copy(obuf.at[b], out_hbm.at[0], wsem.at[b]).wait()

  for c in range(NBUF - 1):
    start_g(c, c)

  @pl.loop(0, BPW, step=NBUF)
  def _(c0):
    for b in range(NBUF):
      c = c0 + b

      @pl.when(c >= 1)
      def _():
        wait_w((c + NBUF - 1) % NBUF)

      @pl.when(c + NBUF - 1 < BPW)
      def _():
        start_g(c + NBUF - 1, (c + NBUF - 1) % NBUF)

      wait_g(b)
      start_w(c, b)

  wait_w((BPW - 1) % NBUF)


@functools.partial(jax.jit, donate_argnums=())
def _run(idx, x_table, y_table):
  mesh = plsc.VectorSubcoreMesh(core_axis_name="c", subcore_axis_name="s")
  kfn = pl.kernel(
      _body,
      out_type=jax.ShapeDtypeStruct((B, S, 2 * HALF), jnp.float32),
      mesh=mesh,
      scratch_types=[
          pltpu.VMEM((PAIRS,), jnp.int32),
          pltpu.VMEM((2 * XOFF,), jnp.int32),
          pltpu.VMEM((NBUF, S, 2 * HALF), jnp.float32),
          pltpu.SemaphoreType.DMA((NBUF,)),
          pltpu.SemaphoreType.DMA((NBUF,)),
      ],
      compiler_params=pltpu.CompilerParams(needs_layout_passes=False),
  )
  return kfn(idx, x_table, y_table)


def kernel(coords, x_table, y_table):
  idx = jnp.asarray(coords, jnp.int32).reshape(NW, PAIRS)
  return _run(idx, x_table, y_table)


# speedup vs baseline: 4.4361x; 1.0028x over previous
import functools

import jax
import jax.numpy as jnp
from jax import lax
from jax.experimental import pallas as pl
from jax.experimental.pallas import tpu as pltpu
from jax.experimental.pallas import tpu_sc as plsc

HALF = 128
B, S = 4096, 50
N = B * S
NC, NS = 2, 16
NW = NC * NS
BPW = B // NW
ROWS_PER_W = BPW * S
STRIDE = 64
XOFF = BPW * STRIDE
NBUF = 4
PAIRS = 2 * ROWS_PER_W


def _body(idx_hbm, x_hbm, y_hbm, out_hbm, idx_all, idx_u, obuf, gsem, wsem):
  wid = lax.axis_index("s") * NC + lax.axis_index("c")
  pltpu.sync_copy(idx_hbm.at[wid], idx_all)
  lanes = lax.iota(jnp.int32, 16)

  @pl.loop(0, BPW)
  def _(c):
    for p in range(4):
      g = c * S + p * 16 + lanes
      g = jnp.minimum(g, ROWS_PER_W - 1)
      idx_u[pl.ds(c * STRIDE + p * 16, 16)] = plsc.load_gather(
          idx_all, [2 * g])
      idx_u[pl.ds(XOFF + c * STRIDE + p * 16, 16)] = plsc.load_gather(
          idx_all, [2 * g + 1])

  def start_g(c, b):
    pltpu.async_copy(x_hbm.at[idx_u.at[pl.ds(c * STRIDE, S)]],
                     obuf.at[b, :, pl.ds(0, HALF)], gsem.at[b])
    pltpu.async_copy(y_hbm.at[idx_u.at[pl.ds(XOFF + c * STRIDE, S)]],
                     obuf.at[b, :, pl.ds(HALF, HALF)], gsem.at[b])

  def wait_g(b):
    pltpu.make_async_copy(x_hbm.at[idx_u.at[pl.ds(0, S)]],
                          obuf.at[b, :, pl.ds(0, HALF)], gsem.at[b]).wait()
    pltpu.make_async_copy(y_hbm.at[idx_u.at[pl.ds(0, S)]],
                          obuf.at[b, :, pl.ds(HALF, HALF)], gsem.at[b]).wait()

  def start_w(c, b):
    pltpu.async_copy(obuf.at[b], out_hbm.at[wid * BPW + c], wsem.at[b])

  def wait_w(b):
    pltpu.make_async_copy(obuf.at[b], out_hbm.at[0], wsem.at[b]).wait()

  for c in range(NBUF - 1):
    start_g(c, c)

  @pl.loop(0, BPW, step=NBUF)
  def _(c0):
    for b in range(NBUF):
      c = c0 + b

      @pl.when(c >= 1)
      def _():
        wait_w((c + NBUF - 1) % NBUF)

      @pl.when(c + NBUF - 1 < BPW)
      def _():
        start_g(c + NBUF - 1, (c + NBUF - 1) % NBUF)

      wait_g(b)
      start_w(c, b)

  wait_w((BPW - 1) % NBUF)


@functools.partial(jax.jit, donate_argnums=())
def _run(idx, x_table, y_table):
  mesh = plsc.VectorSubcoreMesh(core_axis_name="c", subcore_axis_name="s")
  kfn = pl.kernel(
      _body,
      out_type=jax.ShapeDtypeStruct((B, S, 2 * HALF), jnp.float32),
      mesh=mesh,
      scratch_types=[
          pltpu.VMEM((PAIRS,), jnp.int32),
          pltpu.VMEM((2 * XOFF,), jnp.int32),
          pltpu.VMEM((NBUF, S, 2 * HALF), jnp.float32),
          pltpu.SemaphoreType.DMA((NBUF,)),
          pltpu.SemaphoreType.DMA((NBUF,)),
      ],
      compiler_params=pltpu.CompilerParams(
          needs_layout_passes=False, skip_device_barrier=True),
  )
  return kfn(idx, x_table, y_table)


def kernel(coords, x_table, y_table):
  idx = jnp.asarray(coords, jnp.int32).reshape(NW, PAIRS)
  return _run(idx, x_table, y_table)


# speedup vs baseline: 6.7341x; 1.5180x over previous
import functools

import jax
import jax.numpy as jnp
from jax import lax
from jax.experimental import pallas as pl
from jax.experimental.pallas import tpu as pltpu
from jax.experimental.pallas import tpu_sc as plsc

HALF = 128
B, S = 4096, 50
N = B * S
NC, NS = 2, 16
NW = NC * NS
BPW = B // NW
ROWS_PER_W = BPW * S
XOFF = S * BPW
NBUF = 2
PAIRS = 2 * ROWS_PER_W


def _body(idx_hbm, x_hbm, y_hbm, out_hbm, idx_all, idx_u, obuf, gsem, wsem):
  wid = lax.axis_index("s") * NC + lax.axis_index("c")
  pltpu.sync_copy(idx_hbm.at[wid], idx_all)
  lanes = lax.iota(jnp.int32, 16)

  @pl.loop(0, S)
  def _(s):
    for p in range(BPW // 16):
      g = (p * 16 + lanes) * S + s
      idx_u[pl.ds(s * BPW + p * 16, 16)] = plsc.load_gather(idx_all, [2 * g])
      idx_u[pl.ds(XOFF + s * BPW + p * 16, 16)] = plsc.load_gather(
          idx_all, [2 * g + 1])

  def start_g(s, b):
    pltpu.async_copy(x_hbm.at[idx_u.at[pl.ds(s * BPW, BPW)]],
                     obuf.at[b, :, pl.ds(0, HALF)], gsem.at[b])
    pltpu.async_copy(y_hbm.at[idx_u.at[pl.ds(XOFF + s * BPW, BPW)]],
                     obuf.at[b, :, pl.ds(HALF, HALF)], gsem.at[b])

  def wait_g(b):
    pltpu.make_async_copy(x_hbm.at[idx_u.at[pl.ds(0, BPW)]],
                          obuf.at[b, :, pl.ds(0, HALF)], gsem.at[b]).wait()
    pltpu.make_async_copy(y_hbm.at[idx_u.at[pl.ds(0, BPW)]],
                          obuf.at[b, :, pl.ds(HALF, HALF)], gsem.at[b]).wait()

  def start_w(s, b):
    pltpu.async_copy(obuf.at[b], out_hbm.at[s, pl.ds(wid * BPW, BPW)],
                     wsem.at[b])

  def wait_w(b):
    pltpu.make_async_copy(obuf.at[b], out_hbm.at[0, pl.ds(0, BPW)],
                          wsem.at[b]).wait()

  start_g(0, 0)

  @pl.loop(0, S, step=2)
  def _(s0):
    for b in range(2):
      s = s0 + b

      @pl.when(s >= 1)
      def _():
        wait_w(1 - b)

      @pl.when(s + 1 < S)
      def _():
        start_g(s + 1, 1 - b)

      wait_g(b)
      start_w(s, b)

  wait_w((S - 1) % 2)


@functools.partial(jax.jit, donate_argnums=())
def _run(idx, x_table, y_table):
  mesh = plsc.VectorSubcoreMesh(core_axis_name="c", subcore_axis_name="s")
  kfn = pl.kernel(
      _body,
      out_type=jax.ShapeDtypeStruct((S, B, 2 * HALF), jnp.float32),
      mesh=mesh,
      scratch_types=[
          pltpu.VMEM((PAIRS,), jnp.int32),
          pltpu.VMEM((PAIRS,), jnp.int32),
          pltpu.VMEM((NBUF, BPW, 2 * HALF), jnp.float32),
          pltpu.SemaphoreType.DMA((NBUF,)),
          pltpu.SemaphoreType.DMA((NBUF,)),
      ],
      compiler_params=pltpu.CompilerParams(needs_layout_passes=False),
  )
  return kfn(idx, x_table, y_table)


def kernel(coords, x_table, y_table):
  idx = jnp.asarray(coords, jnp.int32).reshape(NW, PAIRS)
  out = _run(idx, x_table, y_table)
  return jnp.transpose(out, (1, 0, 2))
